# baseline (speedup 1.0000x reference)
import functools

import jax
import jax.numpy as jnp
from jax import lax
from jax.experimental import pallas as pl
from jax.experimental.pallas import tpu as pltpu
from jax.experimental.pallas import tpu_sc as plsc

B = 16384
D = 64
H1 = 64
H2 = 32
NU = 1000000

NC = 2
NS = 16
NW = NC * NS
B_PER_W = B // NW

BLKU = 16384
GRID = -(-NU // BLKU)


def _proj_body(tT_ref, w1_ref, b1_ref, w2_ref, b2_ref, o_ref):
    w = jnp.dot(w1_ref[...], w2_ref[...], preferred_element_type=jnp.float32)
    b = jnp.dot(b1_ref[...], w2_ref[...], preferred_element_type=jnp.float32)
    y = (
        jax.lax.dot_general(
            tT_ref[...], w, (((0,), (0,)), ((), ())),
            preferred_element_type=jnp.float32,
        )
        + b + b2_ref[...]
    )
    def to_bf16_bits(v):
        bits = jax.lax.bitcast_convert_type(v, jnp.int32)
        rnd = bits + 0x7FFF + (jax.lax.shift_right_logical(bits, 16) & 1)
        return jax.lax.shift_right_logical(rnd, 16)

    r0 = to_bf16_bits(y[:, : H2 // 2])
    r1 = to_bf16_bits(y[:, H2 // 2 :])
    o_ref[...] = r0 | jax.lax.shift_left(r1, 16)


_tc_project = pl.pallas_call(
    _proj_body,
    grid=(GRID,),
    in_specs=[
        pl.BlockSpec((D, BLKU), lambda i: (0, i)),
        pl.BlockSpec((D, H1), lambda i: (0, 0)),
        pl.BlockSpec((1, H1), lambda i: (0, 0)),
        pl.BlockSpec((H1, H2), lambda i: (0, 0)),
        pl.BlockSpec((1, H2), lambda i: (0, 0)),
    ],
    out_specs=pl.BlockSpec((BLKU, H2 // 2), lambda i: (i, 0)),
    out_shape=jax.ShapeDtypeStruct((NU, H2 // 2), jnp.int32),
    compiler_params=pltpu.CompilerParams(vmem_limit_bytes=100 * 1024 * 1024),
)


def _gather_body(idx_hbm, tp_hbm, out_hbm, idx_v, rows_v, sem):
    wid = lax.axis_index("s") * NC + lax.axis_index("c")
    base = wid * B_PER_W
    pltpu.sync_copy(idx_hbm.at[wid], idx_v)

    def issue(g, _):
        v = idx_v[pl.ds(g * 16, 16)]
        for j in range(16):
            pltpu.async_copy(
                tp_hbm.at[pl.ds(v[j], 1)],
                rows_v.at[pl.ds(g * 16 + j, 1)],
                sem,
            )
        return 0

    lax.fori_loop(0, B_PER_W // 16, issue, 0)

    def drain(i, _):
        pltpu.make_async_copy(
            tp_hbm.at[pl.ds(0, 1)], rows_v.at[pl.ds(0, 1)], sem
        ).wait()
        return 0

    lax.fori_loop(0, B_PER_W, drain, 0)
    pltpu.sync_copy(rows_v, out_hbm.at[pl.ds(base, B_PER_W)])


@functools.cache
def _sc_gather():
    return pl.kernel(
        _gather_body,
        out_type=jax.ShapeDtypeStruct((B, H2 // 2), jnp.int32),
        mesh=plsc.VectorSubcoreMesh(core_axis_name="c", subcore_axis_name="s"),
        scratch_types=[
            pltpu.VMEM((B_PER_W,), jnp.int32),
            pltpu.VMEM((B_PER_W, H2 // 2), jnp.int32),
            pltpu.SemaphoreType.DMA,
        ],
    )


def kernel(user_id, table, W1, b1, W2, b2):
    idx = user_id.astype(jnp.int32).reshape(NW, B_PER_W)
    tableT = table.T
    tp = _tc_project(tableT, W1, b1.reshape(1, H1), W2, b2.reshape(1, H2))
    x16 = _sc_gather()(idx, tp)
    xb = jax.lax.bitcast_convert_type(x16, jnp.bfloat16)
    out = xb.transpose(0, 2, 1).reshape(B, H2)
    return out.astype(jnp.float32)


# speedup vs baseline: 1.1312x; 1.1312x over previous
import functools

import jax
import jax.numpy as jnp
from jax import lax
from jax.experimental import pallas as pl
from jax.experimental.pallas import tpu as pltpu
from jax.experimental.pallas import tpu_sc as plsc

B = 16384
D = 64
H1 = 64
H2 = 32
NU = 1000000

NC = 2
NS = 16
NW = NC * NS
B_PER_W = B // NW

BLKU = 32768
GRID = -(-NU // BLKU)


def _proj_body(tT_ref, w1_ref, b1_ref, w2_ref, b2_ref, o_ref):
    w = jnp.dot(w1_ref[...], w2_ref[...], preferred_element_type=jnp.float32)
    b = jnp.dot(b1_ref[...], w2_ref[...], preferred_element_type=jnp.float32)
    o_ref[...] = (
        jax.lax.dot_general(
            tT_ref[...], w, (((0,), (0,)), ((), ())),
            preferred_element_type=jnp.float32,
        )
        + b + b2_ref[...]
    )


_tc_project = pl.pallas_call(
    _proj_body,
    grid=(GRID,),
    in_specs=[
        pl.BlockSpec((D, BLKU), lambda i: (0, i)),
        pl.BlockSpec((D, H1), lambda i: (0, 0)),
        pl.BlockSpec((1, H1), lambda i: (0, 0)),
        pl.BlockSpec((H1, H2), lambda i: (0, 0)),
        pl.BlockSpec((1, H2), lambda i: (0, 0)),
    ],
    out_specs=pl.BlockSpec((BLKU, H2), lambda i: (i, 0)),
    out_shape=jax.ShapeDtypeStruct((NU, H2), jnp.float32),
    compiler_params=pltpu.CompilerParams(vmem_limit_bytes=56 * 1024 * 1024),
)


def _gather_body(idx_hbm, tp_hbm, out_hbm, idx_v, rows_v, sem):
    wid = lax.axis_index("s") * NC + lax.axis_index("c")
    base = wid * B_PER_W
    pltpu.sync_copy(idx_hbm.at[wid], idx_v)

    def issue(g, _):
        v = idx_v[pl.ds(g * 16, 16)]
        for j in range(16):
            pltpu.async_copy(
                tp_hbm.at[pl.ds(v[j], 1)],
                rows_v.at[pl.ds(g * 16 + j, 1)],
                sem,
            )
        return 0

    lax.fori_loop(0, B_PER_W // 16, issue, 0)

    def drain(i, _):
        pltpu.make_async_copy(
            tp_hbm.at[pl.ds(0, 1)], rows_v.at[pl.ds(0, 1)], sem
        ).wait()
        return 0

    lax.fori_loop(0, B_PER_W, drain, 0)
    pltpu.sync_copy(rows_v, out_hbm.at[pl.ds(base, B_PER_W)])


@functools.cache
def _sc_gather():
    return pl.kernel(
        _gather_body,
        out_type=jax.ShapeDtypeStruct((B, H2), jnp.float32),
        mesh=plsc.VectorSubcoreMesh(core_axis_name="c", subcore_axis_name="s"),
        scratch_types=[
            pltpu.VMEM((B_PER_W,), jnp.int32),
            pltpu.VMEM((B_PER_W, H2), jnp.float32),
            pltpu.SemaphoreType.DMA,
        ],
    )


def kernel(user_id, table, W1, b1, W2, b2):
    idx = user_id.astype(jnp.int32).reshape(NW, B_PER_W)
    tableT = table.T
    tp = _tc_project(tableT, W1, b1.reshape(1, H1), W2, b2.reshape(1, H2))
    return _sc_gather()(idx, tp)
